# Initial kernel scaffold; baseline (speedup 1.0000x reference)
import jax
import jax.numpy as jnp
from jax.experimental import pallas as pl


def kernel(token_ids, W):
    raise NotImplementedError("write your pallas kernel here")


# speedup vs baseline: 1.1026x; 1.1026x over previous
import functools

import jax
import jax.numpy as jnp
from jax import lax
from jax.experimental import pallas as pl
from jax.experimental.pallas import tpu as pltpu
from jax.experimental.pallas import tpu_sc as plsc

_EMB_DIM = 32
_NTOK = 16384 * 50
_NC, _NS = 2, 16
_NW = _NC * _NS
_B_PER_W = _NTOK // _NW
_CHUNK = 1600
_NCHUNK = _B_PER_W // _CHUNK


def _make_gather(n_emb):
    mesh = plsc.VectorSubcoreMesh(core_axis_name="c", subcore_axis_name="s")

    @functools.partial(
        pl.kernel,
        mesh=mesh,
        out_type=jax.ShapeDtypeStruct((_NTOK, _EMB_DIM), jnp.float32),
        scratch_types=[
            pltpu.VMEM((_CHUNK,), jnp.int32),
            pltpu.VMEM((_CHUNK, _EMB_DIM), jnp.float32),
            pltpu.SemaphoreType.DMA,
        ],
        compiler_params=pltpu.CompilerParams(use_tc_tiling_on_sc=False),
    )
    def k(idx_hbm, table_hbm, out_hbm, idx_v, rows_v, sem):
        wid = lax.axis_index("s") * _NC + lax.axis_index("c")
        base = wid * _B_PER_W

        def body(i, carry):
            off = base + i * _CHUNK
            pltpu.sync_copy(idx_hbm.at[pl.ds(off, _CHUNK)], idx_v)
            pltpu.async_copy(table_hbm.at[idx_v], rows_v, sem).wait()
            pltpu.sync_copy(rows_v, out_hbm.at[pl.ds(off, _CHUNK)])
            return carry

        lax.fori_loop(0, _NCHUNK, body, 0)

    return k


_gather = _make_gather(1_000_000)


def kernel(token_ids, W):
    idx = token_ids.reshape(-1).astype(jnp.int32)
    out = _gather(idx, W)
    return out.reshape(token_ids.shape + (_EMB_DIM,))


# speedup vs baseline: 1.1131x; 1.0095x over previous
import functools

import jax
import jax.numpy as jnp
from jax import lax
from jax.experimental import pallas as pl
from jax.experimental.pallas import tpu as pltpu
from jax.experimental.pallas import tpu_sc as plsc

_EMB_DIM = 32
_NTOK = 16384 * 50
_NC, _NS = 2, 16
_NW = _NC * _NS
_B_PER_W = _NTOK // _NW
_CHUNK = 1280
_NCHUNK = _B_PER_W // _CHUNK


def _make_gather():
    mesh = plsc.VectorSubcoreMesh(core_axis_name="c", subcore_axis_name="s")

    @functools.partial(
        pl.kernel,
        mesh=mesh,
        out_type=jax.ShapeDtypeStruct((_NTOK, _EMB_DIM), jnp.float32),
        scratch_types=[
            pltpu.VMEM((_B_PER_W,), jnp.int32),
            pltpu.VMEM((_CHUNK, _EMB_DIM), jnp.float32),
            pltpu.VMEM((_CHUNK, _EMB_DIM), jnp.float32),
            pltpu.SemaphoreType.DMA,
            pltpu.SemaphoreType.DMA,
            pltpu.SemaphoreType.DMA,
            pltpu.SemaphoreType.DMA,
        ],
        compiler_params=pltpu.CompilerParams(use_tc_tiling_on_sc=False),
    )
    def k(idx_hbm, table_hbm, out_hbm, idx_v, rows0, rows1, sg0, sg1, so0, so1):
        wid = lax.axis_index("s") * _NC + lax.axis_index("c")
        base = wid * _B_PER_W
        rows = (rows0, rows1)
        sg = (sg0, sg1)
        so = (so0, so1)

        pltpu.sync_copy(idx_hbm.at[pl.ds(base, _B_PER_W)], idx_v)

        gathers = [None, None]
        outs = [None, None]
        gathers[0] = pltpu.async_copy(
            table_hbm.at[idx_v.at[pl.ds(0, _CHUNK)]], rows[0], sg[0])
        for i in range(_NCHUNK):
            b = i & 1
            nb = 1 - b
            if i + 1 < _NCHUNK:
                if outs[nb] is not None:
                    outs[nb].wait()
                gathers[nb] = pltpu.async_copy(
                    table_hbm.at[idx_v.at[pl.ds((i + 1) * _CHUNK, _CHUNK)]],
                    rows[nb], sg[nb])
            gathers[b].wait()
            outs[b] = pltpu.async_copy(
                rows[b], out_hbm.at[pl.ds(base + i * _CHUNK, _CHUNK)], so[b])
        for c in outs:
            if c is not None:
                c.wait()

    return k


_gather = _make_gather()


def kernel(token_ids, W):
    idx = token_ids.reshape(-1).astype(jnp.int32)
    out = _gather(idx, W)
    return out.reshape(token_ids.shape + (_EMB_DIM,))


# speedup vs baseline: 1.5118x; 1.3582x over previous
import functools

import jax
import jax.numpy as jnp
from jax import lax
from jax.experimental import pallas as pl
from jax.experimental.pallas import tpu as pltpu
from jax.experimental.pallas import tpu_sc as plsc

_DIM = 32
_SEQ = 50
_BATCH = 16384
_NW = 32
_BPW = _BATCH // _NW
_BLK = 256
_NIT = _SEQ * (_BPW // _BLK)


def _make_lookup():
    mesh = plsc.VectorSubcoreMesh(core_axis_name="c", subcore_axis_name="s")

    @functools.partial(
        pl.kernel,
        mesh=mesh,
        out_type=jax.ShapeDtypeStruct((_SEQ, _DIM, _BATCH), jnp.float32),
        scratch_types=[
            pltpu.VMEM((_BLK,), jnp.int32),
            pltpu.VMEM((_BLK,), jnp.int32),
            pltpu.VMEM((_BLK,), jnp.int32),
            pltpu.VMEM((_BLK,), jnp.int32),
            pltpu.VMEM((_BLK,), jnp.int32),
            pltpu.VMEM((_BLK, 128), jnp.float32),
            pltpu.VMEM((_BLK, 128), jnp.float32),
            pltpu.VMEM((_DIM, _BLK), jnp.float32),
            pltpu.SemaphoreType.DMA,
            pltpu.SemaphoreType.DMA,
        ],
        compiler_params=pltpu.CompilerParams(
            use_tc_tiling_on_sc=True, needs_layout_passes=False),
    )
    def k(idxT, W2, out, idx_v, gi0, gi1, cb0, cb1, rows0, rows1, ob, sg0, sg1):
        wid = lax.axis_index("s") * 2 + lax.axis_index("c")
        b0 = wid * _BPW
        gi = (gi0, gi1)
        cb = (cb0, cb1)
        rows = (rows0, rows1)
        sg = (sg0, sg1)
        iota = lax.iota(jnp.int32, 16)

        def col0_of(i):
            return b0 + (i & 1) * _BLK

        def stage_a(i, b):
            s = i >> 1
            c0 = col0_of(i)
            pltpu.sync_copy(idxT.at[s, pl.ds(c0, _BLK)], idx_v)
            for j in range(_BLK // 16):
                v = idx_v[pl.ds(j * 16, 16)]
                gi[b][pl.ds(j * 16, 16)] = lax.shift_right_logical(v, 2)
                cb[b][pl.ds(j * 16, 16)] = lax.shift_left(v & 3, 5)
            pltpu.async_copy(W2.at[gi[b]], rows[b], sg[b])

        def stage_b(i, b):
            s = i >> 1
            c0 = col0_of(i)
            pltpu.make_async_copy(W2.at[pl.ds(0, _BLK)], rows[b], sg[b]).wait()

            def extract(jj, carry):
                jvec = jj * 16 + iota
                cvec = cb[b][pl.ds(jj * 16, 16)]
                for f in range(_DIM):
                    ob[f, pl.ds(jj * 16, 16)] = plsc.load_gather(
                        rows[b], [jvec, cvec + f])
                return carry

            lax.fori_loop(0, _BLK // 16, extract, 0)
            pltpu.sync_copy(ob, out.at[s, :, pl.ds(c0, _BLK)])

        stage_a(0, 0)

        def body(i2, carry):
            i = i2 * 2
            stage_a(i + 1, 1)
            stage_b(i, 0)
            stage_a(i + 2, 0)
            stage_b(i + 1, 1)
            return carry

        lax.fori_loop(0, _NIT // 2 - 1, body, 0)
        stage_a(_NIT - 1, 1)
        stage_b(_NIT - 2, 0)
        stage_b(_NIT - 1, 1)

    return k


_lookup = _make_lookup()


def kernel(token_ids, W):
    idxT = token_ids.T
    W2 = W.reshape(250000, 128)
    out3 = _lookup(idxT, W2)
    return out3.transpose(2, 0, 1)
